# baseline (speedup 1.0000x reference)
import functools

import jax
import jax.numpy as jnp
from jax import lax
from jax.experimental import pallas as pl
from jax.experimental.pallas import tpu as pltpu
from jax.experimental.pallas import tpu_sc as plsc

N_NODES = 100000
N_EDGES = 3200000
NC = 2
NS = 16
LANES = 16
CH = 128

N_PAD = 100352
E_PAD = 3276800
RPT = N_PAD // NS
BLK_W = 40
BLK_E = 32
BLK_N = 80
CPT_W = E_PAD // NS // CH
CPT_E = E_PAD // (NC * NS) // CH
NBLK_W = CPT_W // BLK_W
NBLK_E = CPT_E // BLK_E
NBLK_N = CPT_E // BLK_N
RING = 8
LOOK = 4


ZR = 64


def _zero_rows(zb):
    z = jnp.zeros((LANES,), jnp.float32)
    for i in range(ZR):
        zb[i, :] = z


def _zero_flat(zb, n):
    z = jnp.zeros((LANES,), jnp.float32)
    for i in range(n // LANES):
        zb[pl.ds(i * LANES, LANES)] = z


def _pipe_block(y_ref, sblk, dblk, ring, acc, gsem, ssem, nch):
    lag = RING - LOOK

    def g_start(c, slot):
        pltpu.async_copy(y_ref.at[sblk.at[c]], ring.at[slot], gsem)

    def g_wait(c, slot):
        pltpu.make_async_copy(y_ref.at[sblk.at[c]], ring.at[slot], gsem).wait()

    def s_start(c, slot):
        pltpu.async_copy(ring.at[slot], acc.at[dblk.at[c]], ssem, add=True)

    def s_wait(c, slot):
        pltpu.make_async_copy(ring.at[slot], acc.at[dblk.at[c]], ssem).wait()

    for u in range(LOOK):
        g_start(u, u)

    def body(t, _):
        for u in range(RING):
            c = RING * t + u
            g_wait(c, u)
            s_start(c, u)

            @pl.when(c >= lag)
            def _():
                s_wait(c - lag, (u - lag) % RING)

            @pl.when(c + LOOK < nch)
            def _():
                g_start(c + LOOK, (u + LOOK) % RING)

        return 0

    lax.fori_loop(0, nch // RING, body, 0)
    for k in range(lag):
        c = nch - lag + k
        s_wait(c, c % RING)


@functools.partial(
    pl.kernel,
    out_type=jax.ShapeDtypeStruct((NC, N_PAD), jnp.float32),
    mesh=plsc.VectorSubcoreMesh(core_axis_name="c", subcore_axis_name="s"),
    compiler_params=pltpu.CompilerParams(use_tc_tiling_on_sc=False),
    scratch_types=[
        pltpu.VMEM((BLK_E, CH), jnp.int32),
        pltpu.VMEM((CH,), jnp.float32),
        pltpu.VMEM((2048,), jnp.float32),
        pltpu.VMEM_SHARED((N_PAD,), jnp.float32),
    ],
)
def _sc_deg(dst2d, out, dblk, ones_v, zb, acc):
    cid = lax.axis_index("c")
    sid = lax.axis_index("s")
    _zero_flat(zb, 2048)
    one = jnp.ones((LANES,), jnp.float32)
    for i in range(CH // LANES):
        ones_v[pl.ds(i * LANES, LANES)] = one
    r0 = sid * RPT
    for i in range(3):
        pltpu.sync_copy(zb, acc.at[pl.ds(r0 + i * 2048, 2048)])
    pltpu.sync_copy(zb.at[pl.ds(0, RPT - 3 * 2048)],
                    acc.at[pl.ds(r0 + 3 * 2048, RPT - 3 * 2048)])
    plsc.subcore_barrier()

    c0 = (cid * NS + sid) * CPT_E

    def blk_body(b, _):
        pltpu.sync_copy(dst2d.at[pl.ds(c0 + b * BLK_E, BLK_E)], dblk)

        def inner(j, _):
            pltpu.sync_copy(ones_v, acc.at[dblk.at[j]], add=True)
            return 0

        lax.fori_loop(0, BLK_E, inner, 0)
        return 0

    lax.fori_loop(0, NBLK_E, blk_body, 0)
    plsc.subcore_barrier()
    pltpu.sync_copy(acc.at[pl.ds(r0, RPT)], out.at[cid, pl.ds(r0, RPT)])


@functools.partial(
    pl.kernel,
    out_type=jax.ShapeDtypeStruct((NC, N_PAD, LANES), jnp.float32),
    mesh=plsc.VectorSubcoreMesh(core_axis_name="c", subcore_axis_name="s"),
    compiler_params=pltpu.CompilerParams(use_tc_tiling_on_sc=False),
    scratch_types=[
        pltpu.VMEM((BLK_W, CH), jnp.int32),
        pltpu.VMEM((BLK_W, CH), jnp.int32),
        pltpu.VMEM((RING, CH, LANES), jnp.float32),
        pltpu.VMEM((ZR, LANES), jnp.float32),
        pltpu.VMEM_SHARED((N_PAD, LANES), jnp.float32),
        pltpu.SemaphoreType.DMA,
        pltpu.SemaphoreType.DMA,
    ],
)
def _sc_wide(src2d, dst2d, y_lo, y_hi, out, sblk, dblk, ring, zb, acc,
             gsem, ssem):
    cid = lax.axis_index("c")
    sid = lax.axis_index("s")
    _zero_rows(zb)
    r0 = sid * RPT
    for i in range(RPT // ZR):
        pltpu.sync_copy(zb, acc.at[pl.ds(r0 + i * ZR, ZR)])
    plsc.subcore_barrier()

    c0 = sid * CPT_W

    def blk_body(b, _):
        pltpu.sync_copy(src2d.at[pl.ds(c0 + b * BLK_W, BLK_W)], sblk)
        pltpu.sync_copy(dst2d.at[pl.ds(c0 + b * BLK_W, BLK_W)], dblk)

        @pl.when(cid == 0)
        def _():
            _pipe_block(y_lo, sblk, dblk, ring, acc, gsem, ssem, BLK_W)

        @pl.when(cid == 1)
        def _():
            _pipe_block(y_hi, sblk, dblk, ring, acc, gsem, ssem, BLK_W)

        return 0

    lax.fori_loop(0, NBLK_W, blk_body, 0)
    plsc.subcore_barrier()
    pltpu.sync_copy(acc.at[pl.ds(r0, RPT)], out.at[cid, pl.ds(r0, RPT)])


@functools.partial(
    pl.kernel,
    out_type=jax.ShapeDtypeStruct((NC, N_PAD), jnp.float32),
    mesh=plsc.VectorSubcoreMesh(core_axis_name="c", subcore_axis_name="s"),
    compiler_params=pltpu.CompilerParams(use_tc_tiling_on_sc=False, needs_layout_passes=False),
    scratch_types=[
        pltpu.VMEM((BLK_N * CH,), jnp.int32),
        pltpu.VMEM((BLK_N, CH), jnp.int32),
        pltpu.VMEM((N_NODES,), jnp.float32),
        pltpu.VMEM((2, CH), jnp.float32),
        pltpu.VMEM((2048,), jnp.float32),
        pltpu.VMEM_SHARED((N_PAD,), jnp.float32),
        pltpu.SemaphoreType.DMA,
    ],
)
def _sc_narrow(src_flat, dst2d, y3, out, sflat, dblk, y3v, stage, zb, acc,
               ssem):
    cid = lax.axis_index("c")
    sid = lax.axis_index("s")
    pltpu.sync_copy(y3, y3v)
    _zero_flat(zb, 2048)
    r0 = sid * RPT
    for i in range(3):
        pltpu.sync_copy(zb, acc.at[pl.ds(r0 + i * 2048, 2048)])
    pltpu.sync_copy(zb.at[pl.ds(0, RPT - 3 * 2048)],
                    acc.at[pl.ds(r0 + 3 * 2048, RPT - 3 * 2048)])
    plsc.subcore_barrier()

    e0 = (cid * NS + sid) * CPT_E * CH

    def s_start(c, slot):
        pltpu.async_copy(stage.at[slot], acc.at[dblk.at[c]], ssem, add=True)

    def s_wait(c, slot):
        pltpu.make_async_copy(stage.at[slot], acc.at[dblk.at[c]], ssem).wait()

    def blk_body(b, _):
        pltpu.sync_copy(src_flat.at[pl.ds(e0 + b * BLK_N * CH, BLK_N * CH)],
                        sflat)
        pltpu.sync_copy(dst2d.at[pl.ds(e0 // CH + b * BLK_N, BLK_N)], dblk)

        def inner(t, _):
            for u in range(2):
                c = 2 * t + u

                @pl.when(c >= 2)
                def _():
                    s_wait(c - 2, u)

                for g in range(CH // LANES):
                    idx = sflat[pl.ds(c * CH + g * LANES, LANES)]
                    stage[u, pl.ds(g * LANES, LANES)] = plsc.load_gather(
                        y3v, [idx])
                s_start(c, u)
            return 0

        lax.fori_loop(0, BLK_N // 2, inner, 0)
        s_wait(BLK_N - 2, 0)
        s_wait(BLK_N - 1, 1)
        return 0

    lax.fori_loop(0, NBLK_N, blk_body, 0)
    plsc.subcore_barrier()
    pltpu.sync_copy(acc.at[pl.ds(r0, RPT)], out.at[cid, pl.ds(r0, RPT)])


TBLK = 4000
GRID = N_NODES // TBLK


def _full(shape):
    return pl.BlockSpec(shape, lambda i: tuple(0 for _ in shape))


def _rows(shape):
    return pl.BlockSpec(shape, lambda i: (i,) + tuple(0 for _ in shape[1:]))


def _tc1_body(x_r, degp_r, w1, b1, w2, b2, wc_h, wc_x, dinv_r, ylo_r, yhi_r):
    deg = degp_r[:, 0:1] + degp_r[:, 1:2] + 1.0
    dinv = lax.rsqrt(jnp.maximum(deg, 1e-12))
    h = jnp.maximum(jnp.dot(x_r[...], w1[...],
                            preferred_element_type=jnp.float32) + b1[...], 0.0)
    h = jnp.maximum(jnp.dot(h, w2[...],
                            preferred_element_type=jnp.float32) + b2[...], 0.0)
    y = (jnp.dot(h, wc_h[...], preferred_element_type=jnp.float32)
         + jnp.dot(x_r[...], wc_x[...], preferred_element_type=jnp.float32)) * dinv
    dinv_r[...] = dinv
    ylo_r[...] = y[:, :LANES]
    yhi_r[...] = y[:, LANES:]


_tc1 = pl.pallas_call(
    _tc1_body,
    grid=(GRID,),
    in_specs=[
        _rows((TBLK, 3)),
        _rows((TBLK, 2)),
        _full((3, 16)),
        _full((1, 16)),
        _full((16, 16)),
        _full((1, 16)),
        _full((16, 32)),
        _full((3, 32)),
    ],
    out_specs=[_rows((TBLK, 1)), _rows((TBLK, LANES)), _rows((TBLK, LANES))],
    out_shape=[
        jax.ShapeDtypeStruct((N_NODES, 1), jnp.float32),
        jax.ShapeDtypeStruct((N_NODES, LANES), jnp.float32),
        jax.ShapeDtypeStruct((N_NODES, LANES), jnp.float32),
    ],
)


def _combine_body(acc_r, ylo_r, yhi_r, dinv_r, x_r, bcur, wn_h, wn_x,
                  olo_r, ohi_r):
    dinv = dinv_r[...]
    a_lo = acc_r[0] + ylo_r[...]
    a_hi = acc_r[1] + yhi_r[...]
    o_lo = jnp.maximum(dinv * a_lo + bcur[:, :LANES], 0.0)
    o_hi = jnp.maximum(dinv * a_hi + bcur[:, LANES:], 0.0)
    y = (jnp.dot(o_lo, wn_h[:LANES], preferred_element_type=jnp.float32)
         + jnp.dot(o_hi, wn_h[LANES:], preferred_element_type=jnp.float32)
         + jnp.dot(x_r[...], wn_x[...], preferred_element_type=jnp.float32)) * dinv
    if olo_r is ohi_r:
        olo_r[...] = y
    else:
        olo_r[...] = y[:, :LANES]
        ohi_r[...] = y[:, LANES:]


def _make_combine(wide):
    wout = 32 if wide else 1

    def body(acc_r, ylo_r, yhi_r, dinv_r, x_r, bcur, wn_h, wn_x, *outs):
        if wide:
            _combine_body(acc_r, ylo_r, yhi_r, dinv_r, x_r, bcur, wn_h, wn_x,
                          outs[0], outs[1])
        else:
            _combine_body(acc_r, ylo_r, yhi_r, dinv_r, x_r, bcur, wn_h, wn_x,
                          outs[0], outs[0])

    out_specs = ([_rows((TBLK, LANES)), _rows((TBLK, LANES))] if wide
                 else [_rows((TBLK, 1))])
    out_shape = ([jax.ShapeDtypeStruct((N_NODES, LANES), jnp.float32)] * 2 if wide
                 else [jax.ShapeDtypeStruct((N_NODES, 1), jnp.float32)])
    return pl.pallas_call(
        body,
        grid=(GRID,),
        in_specs=[
            pl.BlockSpec((2, TBLK, LANES), lambda i: (0, i, 0)),
            _rows((TBLK, LANES)),
            _rows((TBLK, LANES)),
            _rows((TBLK, 1)),
            _rows((TBLK, 3)),
            _full((1, 32)),
            _full((32, wout)),
            _full((3, wout)),
        ],
        out_specs=out_specs,
        out_shape=out_shape,
    )


_tc2 = _make_combine(True)
_tc3 = _make_combine(False)


def _tc4_body(accp_r, y3_r, dinv_r, b3, out_r):
    a = accp_r[:, 0:1] + accp_r[:, 1:2] + y3_r[...]
    out_r[...] = dinv_r[...] * a + b3[...]


_tc4 = pl.pallas_call(
    _tc4_body,
    grid=(GRID,),
    in_specs=[
        _rows((TBLK, 2)),
        _rows((TBLK, 1)),
        _rows((TBLK, 1)),
        _full((1, 1)),
    ],
    out_specs=_rows((TBLK, 1)),
    out_shape=jax.ShapeDtypeStruct((N_NODES, 1), jnp.float32),
)


def kernel(x, edge_index, W_fc1, b_fc1, W_fc2, b_fc2, W_c1, b_c1, W_c2, b_c2,
           W_c3, b_c3):
    n = x.shape[0]
    e = edge_index.shape[1]
    pad = E_PAD - e
    src = jnp.concatenate([edge_index[0], jnp.zeros((pad,), jnp.int32)])
    dst = jnp.concatenate([edge_index[1], jnp.full((pad,), n, jnp.int32)])
    src2d = src.reshape(-1, CH)
    dst2d = dst.reshape(-1, CH)

    def padw(w, rows, tgt):
        return jnp.pad(w, ((0, rows - w.shape[0]), (0, tgt - w.shape[1])))

    wc1_h = padw(W_c1[:16], 16, 32)
    wc1_x = padw(W_c1[16:], 3, 32)
    bc1 = jnp.pad(b_c1, (0, 3)).reshape(1, 32)
    wc2_h = padw(W_c2[:29], 32, 32)
    wc2_x = padw(W_c2[29:], 3, 32)
    bc2 = jnp.pad(b_c2, (0, 3)).reshape(1, 32)
    wc3_h = jnp.pad(W_c3[:29], ((0, 3), (0, 0)))
    wc3_x = W_c3[29:]
    b1 = b_fc1.reshape(1, 16)
    b2 = b_fc2.reshape(1, 16)
    b3 = b_c3.reshape(1, 1)

    degp = _sc_deg(dst2d)
    degp_t = degp.T[:N_NODES]
    dinv, y1_lo, y1_hi = _tc1(x, degp_t, W_fc1, b1, W_fc2, b2, wc1_h, wc1_x)
    acc1 = _sc_wide(src2d, dst2d, y1_lo, y1_hi)
    y2_lo, y2_hi = _tc2(acc1, y1_lo, y1_hi, dinv, x, bc1, wc2_h, wc2_x)
    acc2 = _sc_wide(src2d, dst2d, y2_lo, y2_hi)
    (y3,) = _tc3(acc2, y2_lo, y2_hi, dinv, x, bc2, wc3_h, wc3_x)
    acc3 = _sc_narrow(src, dst2d, y3.reshape(-1))
    acc3_t = acc3.T[:N_NODES]
    return _tc4(acc3_t, y3, dinv, b3)


# speedup vs baseline: 1.0450x; 1.0450x over previous
import functools

import jax
import jax.numpy as jnp
from jax import lax
from jax.experimental import pallas as pl
from jax.experimental.pallas import tpu as pltpu
from jax.experimental.pallas import tpu_sc as plsc

N_NODES = 100000
N_EDGES = 3200000
NC = 2
NS = 16
LANES = 16
CH = 128

N_PAD = 100352
E_PAD = 3276800
RPT = N_PAD // NS
BLK_W = 20
BLK_E = 32
BLK_N = 80
CPT_W = E_PAD // NS // CH
CPT_E = E_PAD // (NC * NS) // CH
NBLK_W = CPT_W // BLK_W
NBLK_E = CPT_E // BLK_E
NBLK_N = CPT_E // BLK_N
RING = 10
LOOK = 9


ZR = 64


def _zero_rows(zb):
    z = jnp.zeros((LANES,), jnp.float32)
    for i in range(ZR):
        zb[i, :] = z


def _zero_flat(zb, n):
    z = jnp.zeros((LANES,), jnp.float32)
    for i in range(n // LANES):
        zb[pl.ds(i * LANES, LANES)] = z


def _pipe_block(y_ref, sblk, dblk, ring, acc, gsem, ssem, nch):
    lag = RING - LOOK

    def g_start(c, slot):
        pltpu.async_copy(y_ref.at[sblk.at[c]], ring.at[slot], gsem)

    def g_wait(c, slot):
        pltpu.make_async_copy(y_ref.at[sblk.at[c]], ring.at[slot], gsem).wait()

    def s_start(c, slot):
        pltpu.async_copy(ring.at[slot], acc.at[dblk.at[c]], ssem, add=True)

    def s_wait(c, slot):
        pltpu.make_async_copy(ring.at[slot], acc.at[dblk.at[c]], ssem).wait()

    for u in range(LOOK):
        g_start(u, u)

    def body(t, _):
        for u in range(RING):
            c = RING * t + u
            g_wait(c, u)
            s_start(c, u)

            @pl.when(c >= lag)
            def _():
                s_wait(c - lag, (u - lag) % RING)

            @pl.when(c + LOOK < nch)
            def _():
                g_start(c + LOOK, (u + LOOK) % RING)

        return 0

    lax.fori_loop(0, nch // RING, body, 0)
    for k in range(lag):
        c = nch - lag + k
        s_wait(c, c % RING)


@functools.partial(
    pl.kernel,
    out_type=jax.ShapeDtypeStruct((NC, N_PAD), jnp.float32),
    mesh=plsc.VectorSubcoreMesh(core_axis_name="c", subcore_axis_name="s"),
    compiler_params=pltpu.CompilerParams(use_tc_tiling_on_sc=False),
    scratch_types=[
        pltpu.VMEM((BLK_E, CH), jnp.int32),
        pltpu.VMEM((CH,), jnp.float32),
        pltpu.VMEM((2048,), jnp.float32),
        pltpu.VMEM_SHARED((N_PAD,), jnp.float32),
    ],
)
def _sc_deg(dst2d, out, dblk, ones_v, zb, acc):
    cid = lax.axis_index("c")
    sid = lax.axis_index("s")
    _zero_flat(zb, 2048)
    one = jnp.ones((LANES,), jnp.float32)
    for i in range(CH // LANES):
        ones_v[pl.ds(i * LANES, LANES)] = one
    r0 = sid * RPT
    for i in range(3):
        pltpu.sync_copy(zb, acc.at[pl.ds(r0 + i * 2048, 2048)])
    pltpu.sync_copy(zb.at[pl.ds(0, RPT - 3 * 2048)],
                    acc.at[pl.ds(r0 + 3 * 2048, RPT - 3 * 2048)])
    plsc.subcore_barrier()

    c0 = (cid * NS + sid) * CPT_E

    def blk_body(b, _):
        pltpu.sync_copy(dst2d.at[pl.ds(c0 + b * BLK_E, BLK_E)], dblk)

        def inner(j, _):
            pltpu.sync_copy(ones_v, acc.at[dblk.at[j]], add=True)
            return 0

        lax.fori_loop(0, BLK_E, inner, 0)
        return 0

    lax.fori_loop(0, NBLK_E, blk_body, 0)
    plsc.subcore_barrier()
    pltpu.sync_copy(acc.at[pl.ds(r0, RPT)], out.at[cid, pl.ds(r0, RPT)])


@functools.partial(
    pl.kernel,
    out_type=jax.ShapeDtypeStruct((NC, N_PAD, LANES), jnp.float32),
    mesh=plsc.VectorSubcoreMesh(core_axis_name="c", subcore_axis_name="s"),
    compiler_params=pltpu.CompilerParams(use_tc_tiling_on_sc=False),
    scratch_types=[
        pltpu.VMEM((BLK_W, CH), jnp.int32),
        pltpu.VMEM((BLK_W, CH), jnp.int32),
        pltpu.VMEM((RING, CH, LANES), jnp.float32),
        pltpu.VMEM((ZR, LANES), jnp.float32),
        pltpu.VMEM_SHARED((N_PAD, LANES), jnp.float32),
        pltpu.SemaphoreType.DMA,
        pltpu.SemaphoreType.DMA,
    ],
)
def _sc_wide(src2d, dst2d, y_lo, y_hi, out, sblk, dblk, ring, zb, acc,
             gsem, ssem):
    cid = lax.axis_index("c")
    sid = lax.axis_index("s")
    _zero_rows(zb)
    r0 = sid * RPT
    for i in range(RPT // ZR):
        pltpu.sync_copy(zb, acc.at[pl.ds(r0 + i * ZR, ZR)])
    plsc.subcore_barrier()

    c0 = sid * CPT_W

    def blk_body(b, _):
        pltpu.sync_copy(src2d.at[pl.ds(c0 + b * BLK_W, BLK_W)], sblk)
        pltpu.sync_copy(dst2d.at[pl.ds(c0 + b * BLK_W, BLK_W)], dblk)

        @pl.when(cid == 0)
        def _():
            _pipe_block(y_lo, sblk, dblk, ring, acc, gsem, ssem, BLK_W)

        @pl.when(cid == 1)
        def _():
            _pipe_block(y_hi, sblk, dblk, ring, acc, gsem, ssem, BLK_W)

        return 0

    lax.fori_loop(0, NBLK_W, blk_body, 0)
    plsc.subcore_barrier()
    pltpu.sync_copy(acc.at[pl.ds(r0, RPT)], out.at[cid, pl.ds(r0, RPT)])


@functools.partial(
    pl.kernel,
    out_type=jax.ShapeDtypeStruct((NC, N_PAD), jnp.float32),
    mesh=plsc.VectorSubcoreMesh(core_axis_name="c", subcore_axis_name="s"),
    compiler_params=pltpu.CompilerParams(use_tc_tiling_on_sc=False, needs_layout_passes=False),
    scratch_types=[
        pltpu.VMEM((BLK_N * CH,), jnp.int32),
        pltpu.VMEM((BLK_N, CH), jnp.int32),
        pltpu.VMEM((N_NODES,), jnp.float32),
        pltpu.VMEM((2, CH), jnp.float32),
        pltpu.VMEM((2048,), jnp.float32),
        pltpu.VMEM_SHARED((N_PAD,), jnp.float32),
        pltpu.SemaphoreType.DMA,
    ],
)
def _sc_narrow(src_flat, dst2d, y3, out, sflat, dblk, y3v, stage, zb, acc,
               ssem):
    cid = lax.axis_index("c")
    sid = lax.axis_index("s")
    pltpu.sync_copy(y3, y3v)
    _zero_flat(zb, 2048)
    r0 = sid * RPT
    for i in range(3):
        pltpu.sync_copy(zb, acc.at[pl.ds(r0 + i * 2048, 2048)])
    pltpu.sync_copy(zb.at[pl.ds(0, RPT - 3 * 2048)],
                    acc.at[pl.ds(r0 + 3 * 2048, RPT - 3 * 2048)])
    plsc.subcore_barrier()

    e0 = (cid * NS + sid) * CPT_E * CH

    def s_start(c, slot):
        pltpu.async_copy(stage.at[slot], acc.at[dblk.at[c]], ssem, add=True)

    def s_wait(c, slot):
        pltpu.make_async_copy(stage.at[slot], acc.at[dblk.at[c]], ssem).wait()

    def blk_body(b, _):
        pltpu.sync_copy(src_flat.at[pl.ds(e0 + b * BLK_N * CH, BLK_N * CH)],
                        sflat)
        pltpu.sync_copy(dst2d.at[pl.ds(e0 // CH + b * BLK_N, BLK_N)], dblk)

        def inner(t, _):
            for u in range(2):
                c = 2 * t + u

                @pl.when(c >= 2)
                def _():
                    s_wait(c - 2, u)

                for g in range(CH // LANES):
                    idx = sflat[pl.ds(c * CH + g * LANES, LANES)]
                    stage[u, pl.ds(g * LANES, LANES)] = plsc.load_gather(
                        y3v, [idx])
                s_start(c, u)
            return 0

        lax.fori_loop(0, BLK_N // 2, inner, 0)
        s_wait(BLK_N - 2, 0)
        s_wait(BLK_N - 1, 1)
        return 0

    lax.fori_loop(0, NBLK_N, blk_body, 0)
    plsc.subcore_barrier()
    pltpu.sync_copy(acc.at[pl.ds(r0, RPT)], out.at[cid, pl.ds(r0, RPT)])


TBLK = 4000
GRID = N_NODES // TBLK


def _full(shape):
    return pl.BlockSpec(shape, lambda i: tuple(0 for _ in shape))


def _rows(shape):
    return pl.BlockSpec(shape, lambda i: (i,) + tuple(0 for _ in shape[1:]))


def _tc1_body(x_r, degp_r, w1, b1, w2, b2, wc_h, wc_x, dinv_r, ylo_r, yhi_r):
    deg = degp_r[:, 0:1] + degp_r[:, 1:2] + 1.0
    dinv = lax.rsqrt(jnp.maximum(deg, 1e-12))
    h = jnp.maximum(jnp.dot(x_r[...], w1[...],
                            preferred_element_type=jnp.float32) + b1[...], 0.0)
    h = jnp.maximum(jnp.dot(h, w2[...],
                            preferred_element_type=jnp.float32) + b2[...], 0.0)
    y = (jnp.dot(h, wc_h[...], preferred_element_type=jnp.float32)
         + jnp.dot(x_r[...], wc_x[...], preferred_element_type=jnp.float32)) * dinv
    dinv_r[...] = dinv
    ylo_r[...] = y[:, :LANES]
    yhi_r[...] = y[:, LANES:]


_tc1 = pl.pallas_call(
    _tc1_body,
    grid=(GRID,),
    in_specs=[
        _rows((TBLK, 3)),
        _rows((TBLK, 2)),
        _full((3, 16)),
        _full((1, 16)),
        _full((16, 16)),
        _full((1, 16)),
        _full((16, 32)),
        _full((3, 32)),
    ],
    out_specs=[_rows((TBLK, 1)), _rows((TBLK, LANES)), _rows((TBLK, LANES))],
    out_shape=[
        jax.ShapeDtypeStruct((N_NODES, 1), jnp.float32),
        jax.ShapeDtypeStruct((N_NODES, LANES), jnp.float32),
        jax.ShapeDtypeStruct((N_NODES, LANES), jnp.float32),
    ],
)


def _combine_body(acc_r, ylo_r, yhi_r, dinv_r, x_r, bcur, wn_h, wn_x,
                  olo_r, ohi_r):
    dinv = dinv_r[...]
    a_lo = acc_r[0] + ylo_r[...]
    a_hi = acc_r[1] + yhi_r[...]
    o_lo = jnp.maximum(dinv * a_lo + bcur[:, :LANES], 0.0)
    o_hi = jnp.maximum(dinv * a_hi + bcur[:, LANES:], 0.0)
    y = (jnp.dot(o_lo, wn_h[:LANES], preferred_element_type=jnp.float32)
         + jnp.dot(o_hi, wn_h[LANES:], preferred_element_type=jnp.float32)
         + jnp.dot(x_r[...], wn_x[...], preferred_element_type=jnp.float32)) * dinv
    if olo_r is ohi_r:
        olo_r[...] = y
    else:
        olo_r[...] = y[:, :LANES]
        ohi_r[...] = y[:, LANES:]


def _make_combine(wide):
    wout = 32 if wide else 1

    def body(acc_r, ylo_r, yhi_r, dinv_r, x_r, bcur, wn_h, wn_x, *outs):
        if wide:
            _combine_body(acc_r, ylo_r, yhi_r, dinv_r, x_r, bcur, wn_h, wn_x,
                          outs[0], outs[1])
        else:
            _combine_body(acc_r, ylo_r, yhi_r, dinv_r, x_r, bcur, wn_h, wn_x,
                          outs[0], outs[0])

    out_specs = ([_rows((TBLK, LANES)), _rows((TBLK, LANES))] if wide
                 else [_rows((TBLK, 1))])
    out_shape = ([jax.ShapeDtypeStruct((N_NODES, LANES), jnp.float32)] * 2 if wide
                 else [jax.ShapeDtypeStruct((N_NODES, 1), jnp.float32)])
    return pl.pallas_call(
        body,
        grid=(GRID,),
        in_specs=[
            pl.BlockSpec((2, TBLK, LANES), lambda i: (0, i, 0)),
            _rows((TBLK, LANES)),
            _rows((TBLK, LANES)),
            _rows((TBLK, 1)),
            _rows((TBLK, 3)),
            _full((1, 32)),
            _full((32, wout)),
            _full((3, wout)),
        ],
        out_specs=out_specs,
        out_shape=out_shape,
    )


_tc2 = _make_combine(True)
_tc3 = _make_combine(False)


def _tc4_body(accp_r, y3_r, dinv_r, b3, out_r):
    a = accp_r[:, 0:1] + accp_r[:, 1:2] + y3_r[...]
    out_r[...] = dinv_r[...] * a + b3[...]


_tc4 = pl.pallas_call(
    _tc4_body,
    grid=(GRID,),
    in_specs=[
        _rows((TBLK, 2)),
        _rows((TBLK, 1)),
        _rows((TBLK, 1)),
        _full((1, 1)),
    ],
    out_specs=_rows((TBLK, 1)),
    out_shape=jax.ShapeDtypeStruct((N_NODES, 1), jnp.float32),
)


def kernel(x, edge_index, W_fc1, b_fc1, W_fc2, b_fc2, W_c1, b_c1, W_c2, b_c2,
           W_c3, b_c3):
    n = x.shape[0]
    e = edge_index.shape[1]
    pad = E_PAD - e
    src = jnp.concatenate([edge_index[0], jnp.zeros((pad,), jnp.int32)])
    dst = jnp.concatenate([edge_index[1], jnp.full((pad,), n, jnp.int32)])
    src2d = src.reshape(-1, CH)
    dst2d = dst.reshape(-1, CH)

    def padw(w, rows, tgt):
        return jnp.pad(w, ((0, rows - w.shape[0]), (0, tgt - w.shape[1])))

    wc1_h = padw(W_c1[:16], 16, 32)
    wc1_x = padw(W_c1[16:], 3, 32)
    bc1 = jnp.pad(b_c1, (0, 3)).reshape(1, 32)
    wc2_h = padw(W_c2[:29], 32, 32)
    wc2_x = padw(W_c2[29:], 3, 32)
    bc2 = jnp.pad(b_c2, (0, 3)).reshape(1, 32)
    wc3_h = jnp.pad(W_c3[:29], ((0, 3), (0, 0)))
    wc3_x = W_c3[29:]
    b1 = b_fc1.reshape(1, 16)
    b2 = b_fc2.reshape(1, 16)
    b3 = b_c3.reshape(1, 1)

    degp = _sc_deg(dst2d)
    degp_t = degp.T[:N_NODES]
    dinv, y1_lo, y1_hi = _tc1(x, degp_t, W_fc1, b1, W_fc2, b2, wc1_h, wc1_x)
    acc1 = _sc_wide(src2d, dst2d, y1_lo, y1_hi)
    y2_lo, y2_hi = _tc2(acc1, y1_lo, y1_hi, dinv, x, bc1, wc2_h, wc2_x)
    acc2 = _sc_wide(src2d, dst2d, y2_lo, y2_hi)
    (y3,) = _tc3(acc2, y2_lo, y2_hi, dinv, x, bc2, wc3_h, wc3_x)
    acc3 = _sc_narrow(src, dst2d, y3.reshape(-1))
    acc3_t = acc3.T[:N_NODES]
    return _tc4(acc3_t, y3, dinv, b3)


# speedup vs baseline: 1.0682x; 1.0222x over previous
import functools

import jax
import jax.numpy as jnp
from jax import lax
from jax.experimental import pallas as pl
from jax.experimental.pallas import tpu as pltpu
from jax.experimental.pallas import tpu_sc as plsc

N_NODES = 100000
N_EDGES = 3200000
NC = 2
NS = 16
LANES = 16
CH = 128

N_PAD = 100352
E_PAD = 3276800
RPT = N_PAD // NS
BLK_W = 40
BLK_E = 32
BLK_N = 80
CPT_W = E_PAD // NS // CH
CPT_E = E_PAD // (NC * NS) // CH
NBLK_W = CPT_W // BLK_W
NBLK_E = CPT_E // BLK_E
NBLK_N = CPT_E // BLK_N
RING = 8
LOOK = 7


ZR = 64


def _zero_rows(zb):
    z = jnp.zeros((LANES,), jnp.float32)
    for i in range(ZR):
        zb[i, :] = z


def _zero_flat(zb, n):
    z = jnp.zeros((LANES,), jnp.float32)
    for i in range(n // LANES):
        zb[pl.ds(i * LANES, LANES)] = z


def _pipe_block(y_ref, sblk, dblk, ring, acc, gsem, ssem, nch):
    lag = RING - LOOK

    def g_start(c, slot):
        pltpu.async_copy(y_ref.at[sblk.at[c]], ring.at[slot], gsem)

    def g_wait(c, slot):
        pltpu.make_async_copy(y_ref.at[sblk.at[c]], ring.at[slot], gsem).wait()

    def s_start(c, slot):
        pltpu.async_copy(ring.at[slot], acc.at[dblk.at[c]], ssem, add=True)

    def s_wait(c, slot):
        pltpu.make_async_copy(ring.at[slot], acc.at[dblk.at[c]], ssem).wait()

    for u in range(LOOK):
        g_start(u, u)

    def body(t, _):
        for u in range(RING):
            c = RING * t + u
            g_wait(c, u)
            s_start(c, u)

            @pl.when(c >= lag)
            def _():
                s_wait(c - lag, (u - lag) % RING)

            @pl.when(c + LOOK < nch)
            def _():
                g_start(c + LOOK, (u + LOOK) % RING)

        return 0

    lax.fori_loop(0, nch // RING, body, 0)
    for k in range(lag):
        c = nch - lag + k
        s_wait(c, c % RING)


@functools.partial(
    pl.kernel,
    out_type=jax.ShapeDtypeStruct((NC, N_PAD), jnp.float32),
    mesh=plsc.VectorSubcoreMesh(core_axis_name="c", subcore_axis_name="s"),
    compiler_params=pltpu.CompilerParams(use_tc_tiling_on_sc=False),
    scratch_types=[
        pltpu.VMEM((BLK_E, CH), jnp.int32),
        pltpu.VMEM((CH,), jnp.float32),
        pltpu.VMEM((2048,), jnp.float32),
        pltpu.VMEM_SHARED((N_PAD,), jnp.float32),
    ],
)
def _sc_deg(dst2d, out, dblk, ones_v, zb, acc):
    cid = lax.axis_index("c")
    sid = lax.axis_index("s")
    _zero_flat(zb, 2048)
    one = jnp.ones((LANES,), jnp.float32)
    for i in range(CH // LANES):
        ones_v[pl.ds(i * LANES, LANES)] = one
    r0 = sid * RPT
    for i in range(3):
        pltpu.sync_copy(zb, acc.at[pl.ds(r0 + i * 2048, 2048)])
    pltpu.sync_copy(zb.at[pl.ds(0, RPT - 3 * 2048)],
                    acc.at[pl.ds(r0 + 3 * 2048, RPT - 3 * 2048)])
    plsc.subcore_barrier()

    c0 = (cid * NS + sid) * CPT_E

    def blk_body(b, _):
        pltpu.sync_copy(dst2d.at[pl.ds(c0 + b * BLK_E, BLK_E)], dblk)

        def inner(j, _):
            pltpu.sync_copy(ones_v, acc.at[dblk.at[j]], add=True)
            return 0

        lax.fori_loop(0, BLK_E, inner, 0)
        return 0

    lax.fori_loop(0, NBLK_E, blk_body, 0)
    plsc.subcore_barrier()
    pltpu.sync_copy(acc.at[pl.ds(r0, RPT)], out.at[cid, pl.ds(r0, RPT)])


@functools.partial(
    pl.kernel,
    out_type=jax.ShapeDtypeStruct((NC, N_PAD, LANES), jnp.float32),
    mesh=plsc.VectorSubcoreMesh(core_axis_name="c", subcore_axis_name="s"),
    compiler_params=pltpu.CompilerParams(use_tc_tiling_on_sc=False),
    scratch_types=[
        pltpu.VMEM((BLK_W, CH), jnp.int32),
        pltpu.VMEM((BLK_W, CH), jnp.int32),
        pltpu.VMEM((RING, CH, LANES), jnp.float32),
        pltpu.VMEM((ZR, LANES), jnp.float32),
        pltpu.VMEM_SHARED((N_PAD, LANES), jnp.float32),
        pltpu.SemaphoreType.DMA,
        pltpu.SemaphoreType.DMA,
    ],
)
def _sc_wide(src2d, dst2d, y_lo, y_hi, out, sblk, dblk, ring, zb, acc,
             gsem, ssem):
    cid = lax.axis_index("c")
    sid = lax.axis_index("s")
    _zero_rows(zb)
    r0 = sid * RPT
    for i in range(RPT // ZR):
        pltpu.sync_copy(zb, acc.at[pl.ds(r0 + i * ZR, ZR)])
    plsc.subcore_barrier()

    c0 = sid * CPT_W

    def blk_body(b, _):
        pltpu.sync_copy(src2d.at[pl.ds(c0 + b * BLK_W, BLK_W)], sblk)
        pltpu.sync_copy(dst2d.at[pl.ds(c0 + b * BLK_W, BLK_W)], dblk)

        @pl.when(cid == 0)
        def _():
            _pipe_block(y_lo, sblk, dblk, ring, acc, gsem, ssem, BLK_W)

        @pl.when(cid == 1)
        def _():
            _pipe_block(y_hi, sblk, dblk, ring, acc, gsem, ssem, BLK_W)

        return 0

    lax.fori_loop(0, NBLK_W, blk_body, 0)
    plsc.subcore_barrier()
    pltpu.sync_copy(acc.at[pl.ds(r0, RPT)], out.at[cid, pl.ds(r0, RPT)])


@functools.partial(
    pl.kernel,
    out_type=jax.ShapeDtypeStruct((NC, N_PAD), jnp.float32),
    mesh=plsc.VectorSubcoreMesh(core_axis_name="c", subcore_axis_name="s"),
    compiler_params=pltpu.CompilerParams(use_tc_tiling_on_sc=False, needs_layout_passes=False),
    scratch_types=[
        pltpu.VMEM((BLK_N * CH,), jnp.int32),
        pltpu.VMEM((BLK_N, CH), jnp.int32),
        pltpu.VMEM((N_NODES,), jnp.float32),
        pltpu.VMEM((2, CH), jnp.float32),
        pltpu.VMEM((2048,), jnp.float32),
        pltpu.VMEM_SHARED((N_PAD,), jnp.float32),
        pltpu.SemaphoreType.DMA,
    ],
)
def _sc_narrow(src_flat, dst2d, y3, out, sflat, dblk, y3v, stage, zb, acc,
               ssem):
    cid = lax.axis_index("c")
    sid = lax.axis_index("s")
    pltpu.sync_copy(y3, y3v)
    _zero_flat(zb, 2048)
    r0 = sid * RPT
    for i in range(3):
        pltpu.sync_copy(zb, acc.at[pl.ds(r0 + i * 2048, 2048)])
    pltpu.sync_copy(zb.at[pl.ds(0, RPT - 3 * 2048)],
                    acc.at[pl.ds(r0 + 3 * 2048, RPT - 3 * 2048)])
    plsc.subcore_barrier()

    e0 = (cid * NS + sid) * CPT_E * CH

    def s_start(c, slot):
        pltpu.async_copy(stage.at[slot], acc.at[dblk.at[c]], ssem, add=True)

    def s_wait(c, slot):
        pltpu.make_async_copy(stage.at[slot], acc.at[dblk.at[c]], ssem).wait()

    def blk_body(b, _):
        pltpu.sync_copy(src_flat.at[pl.ds(e0 + b * BLK_N * CH, BLK_N * CH)],
                        sflat)
        pltpu.sync_copy(dst2d.at[pl.ds(e0 // CH + b * BLK_N, BLK_N)], dblk)

        def inner(t, _):
            for u in range(2):
                c = 2 * t + u

                @pl.when(c >= 2)
                def _():
                    s_wait(c - 2, u)

                for g in range(CH // LANES):
                    idx = sflat[pl.ds(c * CH + g * LANES, LANES)]
                    stage[u, pl.ds(g * LANES, LANES)] = plsc.load_gather(
                        y3v, [idx])
                s_start(c, u)
            return 0

        lax.fori_loop(0, BLK_N // 2, inner, 0)
        s_wait(BLK_N - 2, 0)
        s_wait(BLK_N - 1, 1)
        return 0

    lax.fori_loop(0, NBLK_N, blk_body, 0)
    plsc.subcore_barrier()
    pltpu.sync_copy(acc.at[pl.ds(r0, RPT)], out.at[cid, pl.ds(r0, RPT)])


TBLK = 4000
GRID = N_NODES // TBLK


def _full(shape):
    return pl.BlockSpec(shape, lambda i: tuple(0 for _ in shape))


def _rows(shape):
    return pl.BlockSpec(shape, lambda i: (i,) + tuple(0 for _ in shape[1:]))


def _tc1_body(x_r, degp_r, w1, b1, w2, b2, wc_h, wc_x, dinv_r, ylo_r, yhi_r):
    deg = degp_r[:, 0:1] + degp_r[:, 1:2] + 1.0
    dinv = lax.rsqrt(jnp.maximum(deg, 1e-12))
    h = jnp.maximum(jnp.dot(x_r[...], w1[...],
                            preferred_element_type=jnp.float32) + b1[...], 0.0)
    h = jnp.maximum(jnp.dot(h, w2[...],
                            preferred_element_type=jnp.float32) + b2[...], 0.0)
    y = (jnp.dot(h, wc_h[...], preferred_element_type=jnp.float32)
         + jnp.dot(x_r[...], wc_x[...], preferred_element_type=jnp.float32)) * dinv
    dinv_r[...] = dinv
    ylo_r[...] = y[:, :LANES]
    yhi_r[...] = y[:, LANES:]


_tc1 = pl.pallas_call(
    _tc1_body,
    grid=(GRID,),
    in_specs=[
        _rows((TBLK, 3)),
        _rows((TBLK, 2)),
        _full((3, 16)),
        _full((1, 16)),
        _full((16, 16)),
        _full((1, 16)),
        _full((16, 32)),
        _full((3, 32)),
    ],
    out_specs=[_rows((TBLK, 1)), _rows((TBLK, LANES)), _rows((TBLK, LANES))],
    out_shape=[
        jax.ShapeDtypeStruct((N_NODES, 1), jnp.float32),
        jax.ShapeDtypeStruct((N_NODES, LANES), jnp.float32),
        jax.ShapeDtypeStruct((N_NODES, LANES), jnp.float32),
    ],
)


def _combine_body(acc_r, ylo_r, yhi_r, dinv_r, x_r, bcur, wn_h, wn_x,
                  olo_r, ohi_r):
    dinv = dinv_r[...]
    a_lo = acc_r[0] + ylo_r[...]
    a_hi = acc_r[1] + yhi_r[...]
    o_lo = jnp.maximum(dinv * a_lo + bcur[:, :LANES], 0.0)
    o_hi = jnp.maximum(dinv * a_hi + bcur[:, LANES:], 0.0)
    y = (jnp.dot(o_lo, wn_h[:LANES], preferred_element_type=jnp.float32)
         + jnp.dot(o_hi, wn_h[LANES:], preferred_element_type=jnp.float32)
         + jnp.dot(x_r[...], wn_x[...], preferred_element_type=jnp.float32)) * dinv
    if olo_r is ohi_r:
        olo_r[...] = y
    else:
        olo_r[...] = y[:, :LANES]
        ohi_r[...] = y[:, LANES:]


def _make_combine(wide):
    wout = 32 if wide else 1

    def body(acc_r, ylo_r, yhi_r, dinv_r, x_r, bcur, wn_h, wn_x, *outs):
        if wide:
            _combine_body(acc_r, ylo_r, yhi_r, dinv_r, x_r, bcur, wn_h, wn_x,
                          outs[0], outs[1])
        else:
            _combine_body(acc_r, ylo_r, yhi_r, dinv_r, x_r, bcur, wn_h, wn_x,
                          outs[0], outs[0])

    out_specs = ([_rows((TBLK, LANES)), _rows((TBLK, LANES))] if wide
                 else [_rows((TBLK, 1))])
    out_shape = ([jax.ShapeDtypeStruct((N_NODES, LANES), jnp.float32)] * 2 if wide
                 else [jax.ShapeDtypeStruct((N_NODES, 1), jnp.float32)])
    return pl.pallas_call(
        body,
        grid=(GRID,),
        in_specs=[
            pl.BlockSpec((2, TBLK, LANES), lambda i: (0, i, 0)),
            _rows((TBLK, LANES)),
            _rows((TBLK, LANES)),
            _rows((TBLK, 1)),
            _rows((TBLK, 3)),
            _full((1, 32)),
            _full((32, wout)),
            _full((3, wout)),
        ],
        out_specs=out_specs,
        out_shape=out_shape,
    )


_tc2 = _make_combine(True)
_tc3 = _make_combine(False)


def _tc4_body(accp_r, y3_r, dinv_r, b3, out_r):
    a = accp_r[:, 0:1] + accp_r[:, 1:2] + y3_r[...]
    out_r[...] = dinv_r[...] * a + b3[...]


_tc4 = pl.pallas_call(
    _tc4_body,
    grid=(GRID,),
    in_specs=[
        _rows((TBLK, 2)),
        _rows((TBLK, 1)),
        _rows((TBLK, 1)),
        _full((1, 1)),
    ],
    out_specs=_rows((TBLK, 1)),
    out_shape=jax.ShapeDtypeStruct((N_NODES, 1), jnp.float32),
)


def kernel(x, edge_index, W_fc1, b_fc1, W_fc2, b_fc2, W_c1, b_c1, W_c2, b_c2,
           W_c3, b_c3):
    n = x.shape[0]
    e = edge_index.shape[1]
    pad = E_PAD - e
    src = jnp.concatenate([edge_index[0], jnp.zeros((pad,), jnp.int32)])
    dst = jnp.concatenate([edge_index[1], jnp.full((pad,), n, jnp.int32)])
    src2d = src.reshape(-1, CH)
    dst2d = dst.reshape(-1, CH)

    def padw(w, rows, tgt):
        return jnp.pad(w, ((0, rows - w.shape[0]), (0, tgt - w.shape[1])))

    wc1_h = padw(W_c1[:16], 16, 32)
    wc1_x = padw(W_c1[16:], 3, 32)
    bc1 = jnp.pad(b_c1, (0, 3)).reshape(1, 32)
    wc2_h = padw(W_c2[:29], 32, 32)
    wc2_x = padw(W_c2[29:], 3, 32)
    bc2 = jnp.pad(b_c2, (0, 3)).reshape(1, 32)
    wc3_h = jnp.pad(W_c3[:29], ((0, 3), (0, 0)))
    wc3_x = W_c3[29:]
    b1 = b_fc1.reshape(1, 16)
    b2 = b_fc2.reshape(1, 16)
    b3 = b_c3.reshape(1, 1)

    degp = _sc_deg(dst2d)
    degp_t = degp.T[:N_NODES]
    dinv, y1_lo, y1_hi = _tc1(x, degp_t, W_fc1, b1, W_fc2, b2, wc1_h, wc1_x)
    acc1 = _sc_wide(src2d, dst2d, y1_lo, y1_hi)
    y2_lo, y2_hi = _tc2(acc1, y1_lo, y1_hi, dinv, x, bc1, wc2_h, wc2_x)
    acc2 = _sc_wide(src2d, dst2d, y2_lo, y2_hi)
    (y3,) = _tc3(acc2, y2_lo, y2_hi, dinv, x, bc2, wc3_h, wc3_x)
    acc3 = _sc_narrow(src, dst2d, y3.reshape(-1))
    acc3_t = acc3.T[:N_NODES]
    return _tc4(acc3_t, y3, dinv, b3)
